# baseline (speedup 1.0000x reference)
import numpy as np

import jax
import jax.numpy as jnp
from jax import lax
from jax.experimental import pallas as pl
from jax.experimental.pallas import tpu as pltpu
from jax.experimental.pallas import tpu_sc as plsc

E = 16
K = 8
D = 2048
T = 8192
TB = 512
L = 16
NC = 2
NS = 16
SUP = 256
CH = SUP // L

_LANE = np.arange(L)


def _noisy_body(w_ref, b_ref, eps_ref, x_ref, out_ref):
    prod = lax.dot_general(
        w_ref[...], x_ref[...], (((1,), (1,)), ((), ())),
        preferred_element_type=jnp.float32)
    logits = prod[:E, :] + b_ref[:E, :]
    noise_logits = prod[E:, :] + b_ref[E:, :]
    out_ref[...] = logits + eps_ref[...] * jax.nn.softplus(noise_logits)


def _noisy_call(x, Wc, bc, eps, interpret=False):
    return pl.pallas_call(
        _noisy_body,
        grid=(T // TB,),
        in_specs=[
            pl.BlockSpec((2 * E, D), lambda i: (0, 0)),
            pl.BlockSpec((2 * E, 1), lambda i: (0, 0)),
            pl.BlockSpec((E, TB), lambda i: (0, i)),
            pl.BlockSpec((TB, D), lambda i: (i, 0)),
        ],
        out_specs=pl.BlockSpec((E, TB), lambda i: (0, i)),
        out_shape=jax.ShapeDtypeStruct((E, T), jnp.float32),
        interpret=interpret,
    )(Wc, bc, eps, x)


def _lex_gt(va, ia, vb, ib):
    return (va > vb) | ((va == vb) & (ia < ib))


def _substage(v, i, iot, j, flip):
    perm = iot ^ j
    pv = jnp.take_along_axis(v, perm, axis=0)
    pi = jnp.take_along_axis(i, perm, axis=0)
    rank_first = _lex_gt(v, i, pv, pi)
    take_self = rank_first ^ flip
    return jnp.where(take_self, v, pv), jnp.where(take_self, i, pi)


def _bitonic_sort(v, i, iot, descending):
    for k in (2, 4, 8, 16):
        j = k // 2
        while j >= 1:
            upper = (iot & j) != 0
            if k == 16:
                flip = upper if descending else (iot & j) == 0
            else:
                asc_blocks = ((iot & k) != 0) if descending else ((iot & k) == 0)
                flip = upper ^ asc_blocks
            v, i = _substage(v, i, iot, j, flip)
            j //= 2
    return v, i


def _bitonic_merge_desc(v, i, iot):
    for j in (8, 4, 2, 1):
        v, i = _substage(v, i, iot, j, (iot & j) != 0)
    return v, i


def _lane_max(v, iot):
    for sh in (1, 2, 4, 8):
        v = jnp.maximum(v, jnp.take_along_axis(v, iot ^ sh, axis=0))
    return v[0]


def _sc_topk_body(noisy_hbm, router_hbm, idx_hbm, row_v, out_v, idx_v, vals_v):
    cid = lax.axis_index("c")
    sid = lax.axis_index("s")

    @pl.when(sid < E // NC)
    def _():
        e = cid * (E // NC) + sid
        pltpu.sync_copy(noisy_hbm.at[e], row_v)
        iot = lax.iota(jnp.int32, L)
        neg = jnp.float32(-jnp.inf)

        vals_v[...] = jnp.full((L,), neg, jnp.float32)
        idx_v[...] = jnp.zeros((L,), jnp.int32)

        def do_merge(args):
            _, cb = args
            c = row_v[pl.ds(cb * L, L)]
            ci = cb * L + iot
            av, ai = _bitonic_sort(c, ci, iot, descending=False)
            vals = vals_v[...]
            idxs = idx_v[...]
            keep = _lex_gt(vals, idxs, av, ai)
            hv = jnp.where(keep, vals, av)
            hi = jnp.where(keep, idxs, ai)
            nv, ni = _bitonic_merge_desc(hv, hi, iot)
            vals_v[...] = nv
            idx_v[...] = ni
            return nv[K - 1]

        def skip(args):
            return args[0]

        def chunk_step(s, t, kth):
            cb = s * CH + t
            c = row_v[pl.ds(cb * L, L)]
            cmax = _lane_max(c, iot)
            return lax.cond(cmax > kth, do_merge, skip, (kth, cb))

        def sup_step(s, kth):
            def smax_step(t, m):
                return jnp.maximum(m, row_v[pl.ds((s * CH + t) * L, L)])

            m = lax.fori_loop(0, CH, smax_step, jnp.full((L,), neg))
            smax = _lane_max(m, iot)

            def hit(args):
                return lax.fori_loop(
                    0, CH, lambda t, kk: chunk_step(args[1], t, kk), args[0])

            return lax.cond(smax > kth, hit, skip, (kth, s))

        lax.fori_loop(0, T // SUP, sup_step, neg)
        vals = vals_v[...]
        idxs = idx_v[...]

        topmask = iot < K
        ex = jnp.where(topmask, jnp.exp(vals - vals[0]), jnp.float32(0.0))
        s = ex
        for sh in (1, 2, 4, 8):
            s = s + jnp.take_along_axis(s, iot ^ sh, axis=0)
        p = ex / s

        def zstep(i, carry):
            out_v[pl.ds(i * L, L)] = jnp.zeros((L,), jnp.float32)
            return carry

        lax.fori_loop(0, T // L, zstep, 0)

        for k in range(K):
            tk = idxs[k]
            base = (tk >> 4) << 4
            ln = tk & (L - 1)
            vec = out_v[pl.ds(base, L)]
            out_v[pl.ds(base, L)] = jnp.where(iot == ln, p[k], vec)

        pltpu.sync_copy(out_v, router_hbm.at[e])
        pltpu.sync_copy(idx_v, idx_hbm.at[e])


def _sc_topk_call(noisy, interpret=False):
    mesh = plsc.VectorSubcoreMesh(
        core_axis_name="c", subcore_axis_name="s",
        num_cores=NC, num_subcores=NS)
    fn = pl.kernel(
        _sc_topk_body,
        out_type=[
            jax.ShapeDtypeStruct((E, T), jnp.float32),
            jax.ShapeDtypeStruct((E, L), jnp.int32),
        ],
        mesh=mesh,
        scratch_types=[
            pltpu.VMEM((T,), jnp.float32),
            pltpu.VMEM((T,), jnp.float32),
            pltpu.VMEM((L,), jnp.int32),
            pltpu.VMEM((L,), jnp.float32),
        ],
        interpret=interpret,
    )
    return fn(noisy)


def kernel(mh_output, W_route, b_route, W_noise, b_noise, eps):
    x = mh_output.reshape(T, D)
    Wc = jnp.concatenate([W_route, W_noise], axis=0)
    bc = jnp.concatenate([b_route, b_noise])[:, None]
    noisy = _noisy_call(x, Wc, bc, eps)
    router, idx16 = _sc_topk_call(noisy)
    return router, idx16[:, :K]


# speedup vs baseline: 1.0934x; 1.0934x over previous
import numpy as np

import jax
import jax.numpy as jnp
from jax import lax
from jax.experimental import pallas as pl
from jax.experimental.pallas import tpu as pltpu
from jax.experimental.pallas import tpu_sc as plsc

E = 16
K = 8
D = 2048
T = 8192
TB = 1024
L = 16
NC = 2
NS = 16
SUP = 256
CH = SUP // L

_LANE = np.arange(L)


def _noisy_body(w_ref, b_ref, eps_ref, x_ref, out_ref):
    prod = lax.dot_general(
        w_ref[...], x_ref[...], (((1,), (1,)), ((), ())),
        preferred_element_type=jnp.float32)
    logits = prod[:E, :] + b_ref[:E, :]
    noise_logits = prod[E:, :] + b_ref[E:, :]
    out_ref[...] = logits + eps_ref[...] * jax.nn.softplus(noise_logits)


def _noisy_call(x, Wc, bc, eps, interpret=False):
    return pl.pallas_call(
        _noisy_body,
        grid=(T // TB,),
        in_specs=[
            pl.BlockSpec((2 * E, D), lambda i: (0, 0)),
            pl.BlockSpec((2 * E, 1), lambda i: (0, 0)),
            pl.BlockSpec((E, TB), lambda i: (0, i)),
            pl.BlockSpec((TB, D), lambda i: (i, 0)),
        ],
        out_specs=pl.BlockSpec((E, TB), lambda i: (0, i)),
        out_shape=jax.ShapeDtypeStruct((E, T), jnp.float32),
        interpret=interpret,
    )(Wc, bc, eps, x)


def _lex_gt(va, ia, vb, ib):
    return (va > vb) | ((va == vb) & (ia < ib))


def _substage(v, i, iot, j, flip):
    perm = iot ^ j
    pv = jnp.take_along_axis(v, perm, axis=0)
    pi = jnp.take_along_axis(i, perm, axis=0)
    rank_first = _lex_gt(v, i, pv, pi)
    take_self = rank_first ^ flip
    return jnp.where(take_self, v, pv), jnp.where(take_self, i, pi)


def _bitonic_sort(v, i, iot, descending):
    for k in (2, 4, 8, 16):
        j = k // 2
        while j >= 1:
            upper = (iot & j) != 0
            if k == 16:
                flip = upper if descending else (iot & j) == 0
            else:
                asc_blocks = ((iot & k) != 0) if descending else ((iot & k) == 0)
                flip = upper ^ asc_blocks
            v, i = _substage(v, i, iot, j, flip)
            j //= 2
    return v, i


def _bitonic_merge_desc(v, i, iot):
    for j in (8, 4, 2, 1):
        v, i = _substage(v, i, iot, j, (iot & j) != 0)
    return v, i


def _lane_max(v, iot):
    for sh in (1, 2, 4, 8):
        v = jnp.maximum(v, jnp.take_along_axis(v, iot ^ sh, axis=0))
    return v[0]


def _sc_topk_body(noisy_hbm, router_hbm, idx_hbm, row_v, out_v, idx_v, vals_v):
    cid = lax.axis_index("c")
    sid = lax.axis_index("s")

    @pl.when(sid < E // NC)
    def _():
        e = cid * (E // NC) + sid
        pltpu.sync_copy(noisy_hbm.at[e], row_v)
        iot = lax.iota(jnp.int32, L)
        neg = jnp.float32(-jnp.inf)

        vals_v[...] = jnp.full((L,), neg, jnp.float32)
        idx_v[...] = jnp.zeros((L,), jnp.int32)

        def do_merge(args):
            _, cb = args
            c = row_v[pl.ds(cb * L, L)]
            ci = cb * L + iot
            av, ai = _bitonic_sort(c, ci, iot, descending=False)
            vals = vals_v[...]
            idxs = idx_v[...]
            keep = _lex_gt(vals, idxs, av, ai)
            hv = jnp.where(keep, vals, av)
            hi = jnp.where(keep, idxs, ai)
            nv, ni = _bitonic_merge_desc(hv, hi, iot)
            vals_v[...] = nv
            idx_v[...] = ni
            return nv[K - 1]

        def skip(args):
            return args[0]

        def chunk_step(s, t, kth):
            cb = s * CH + t
            c = row_v[pl.ds(cb * L, L)]
            cmax = _lane_max(c, iot)
            return lax.cond(cmax > kth, do_merge, skip, (kth, cb))

        def sup_step(s, kth):
            m = row_v[pl.ds(s * SUP, L)]
            for t in range(1, CH):
                m = jnp.maximum(m, row_v[pl.ds(s * SUP + t * L, L)])
            smax = _lane_max(m, iot)

            def hit(args):
                return lax.fori_loop(
                    0, CH, lambda t, kk: chunk_step(args[1], t, kk), args[0])

            return lax.cond(smax > kth, hit, skip, (kth, s))

        lax.fori_loop(0, T // SUP, sup_step, neg)
        vals = vals_v[...]
        idxs = idx_v[...]

        topmask = iot < K
        ex = jnp.where(topmask, jnp.exp(vals - vals[0]), jnp.float32(0.0))
        s = ex
        for sh in (1, 2, 4, 8):
            s = s + jnp.take_along_axis(s, iot ^ sh, axis=0)
        p = ex / s

        def zstep(i, carry):
            for u in range(8):
                out_v[pl.ds(i * 8 * L + u * L, L)] = jnp.zeros((L,), jnp.float32)
            return carry

        lax.fori_loop(0, T // (8 * L), zstep, 0)

        for k in range(K):
            tk = idxs[k]
            base = (tk >> 4) << 4
            ln = tk & (L - 1)
            vec = out_v[pl.ds(base, L)]
            out_v[pl.ds(base, L)] = jnp.where(iot == ln, p[k], vec)

        pltpu.sync_copy(out_v, router_hbm.at[e])
        pltpu.sync_copy(idx_v, idx_hbm.at[e])


def _sc_topk_call(noisy, interpret=False):
    mesh = plsc.VectorSubcoreMesh(
        core_axis_name="c", subcore_axis_name="s",
        num_cores=NC, num_subcores=NS)
    fn = pl.kernel(
        _sc_topk_body,
        out_type=[
            jax.ShapeDtypeStruct((E, T), jnp.float32),
            jax.ShapeDtypeStruct((E, L), jnp.int32),
        ],
        mesh=mesh,
        scratch_types=[
            pltpu.VMEM((T,), jnp.float32),
            pltpu.VMEM((T,), jnp.float32),
            pltpu.VMEM((L,), jnp.int32),
            pltpu.VMEM((L,), jnp.float32),
        ],
        interpret=interpret,
    )
    return fn(noisy)


def kernel(mh_output, W_route, b_route, W_noise, b_noise, eps):
    x = mh_output.reshape(T, D)
    Wc = jnp.concatenate([W_route, W_noise], axis=0)
    bc = jnp.concatenate([b_route, b_noise])[:, None]
    noisy = _noisy_call(x, Wc, bc, eps)
    router, idx16 = _sc_topk_call(noisy)
    return router, idx16[:, :K]


# speedup vs baseline: 1.1590x; 1.0600x over previous
import numpy as np

import jax
import jax.numpy as jnp
from jax import lax
from jax.experimental import pallas as pl
from jax.experimental.pallas import tpu as pltpu
from jax.experimental.pallas import tpu_sc as plsc

E = 16
K = 8
D = 2048
T = 8192
TB = 1024
L = 16
NC = 2
NS = 16
SUP = 256
CH = SUP // L

_LANE = np.arange(L)


def _noisy_body(w_ref, b_ref, eps_ref, x_ref, out_ref):
    prod = lax.dot_general(
        w_ref[...], x_ref[...], (((1,), (1,)), ((), ())),
        preferred_element_type=jnp.float32)
    logits = prod[:E, :] + b_ref[:E, :]
    noise_logits = prod[E:, :] + b_ref[E:, :]
    out_ref[...] = logits + eps_ref[...] * jax.nn.softplus(noise_logits)


def _noisy_call(x, Wc, bc, eps, interpret=False):
    return pl.pallas_call(
        _noisy_body,
        grid=(T // TB,),
        in_specs=[
            pl.BlockSpec((2 * E, D), lambda i: (0, 0)),
            pl.BlockSpec((2 * E, 1), lambda i: (0, 0)),
            pl.BlockSpec((E, TB), lambda i: (0, i)),
            pl.BlockSpec((TB, D), lambda i: (i, 0)),
        ],
        out_specs=pl.BlockSpec((E, TB), lambda i: (0, i)),
        out_shape=jax.ShapeDtypeStruct((E, T), jnp.float32),
        interpret=interpret,
    )(Wc, bc, eps, x)


def _lex_gt(va, ia, vb, ib):
    return (va > vb) | ((va == vb) & (ia < ib))


def _substage(v, i, iot, j, flip):
    perm = iot ^ j
    pv = jnp.take_along_axis(v, perm, axis=0)
    pi = jnp.take_along_axis(i, perm, axis=0)
    rank_first = _lex_gt(v, i, pv, pi)
    take_self = rank_first ^ flip
    return jnp.where(take_self, v, pv), jnp.where(take_self, i, pi)


def _bitonic_sort(v, i, iot, descending):
    for k in (2, 4, 8, 16):
        j = k // 2
        while j >= 1:
            upper = (iot & j) != 0
            if k == 16:
                flip = upper if descending else (iot & j) == 0
            else:
                asc_blocks = ((iot & k) != 0) if descending else ((iot & k) == 0)
                flip = upper ^ asc_blocks
            v, i = _substage(v, i, iot, j, flip)
            j //= 2
    return v, i


def _bitonic_merge_desc(v, i, iot):
    for j in (8, 4, 2, 1):
        v, i = _substage(v, i, iot, j, (iot & j) != 0)
    return v, i


def _lane_max(v, iot):
    for sh in (1, 2, 4, 8):
        v = jnp.maximum(v, jnp.take_along_axis(v, iot ^ sh, axis=0))
    return v[0]


T2 = T // 2


def _sc_topk_body(noisy_hbm, router_hbm, idx_hbm,
                  row_v, out_v, idx_v, vals_v, pv_v, pi_v, sh_vals, sh_idx):
    cid = lax.axis_index("c")
    sid = lax.axis_index("s")
    half = sid // (E // NC)
    es = sid % (E // NC)
    e = cid * (E // NC) + es

    pltpu.sync_copy(noisy_hbm.at[e, pl.ds(half * T2, T2)], row_v)
    iot = lax.iota(jnp.int32, L)
    neg = jnp.float32(-jnp.inf)
    base_tok = half * T2

    vals_v[...] = jnp.full((L,), neg, jnp.float32)
    idx_v[...] = jnp.zeros((L,), jnp.int32)

    def do_merge(args):
        _, cb = args
        c = row_v[pl.ds(cb * L, L)]
        ci = base_tok + cb * L + iot
        av, ai = _bitonic_sort(c, ci, iot, descending=False)
        vals = vals_v[...]
        idxs = idx_v[...]
        keep = _lex_gt(vals, idxs, av, ai)
        hv = jnp.where(keep, vals, av)
        hi = jnp.where(keep, idxs, ai)
        nv, ni = _bitonic_merge_desc(hv, hi, iot)
        vals_v[...] = nv
        idx_v[...] = ni
        return nv[K - 1]

    def skip(args):
        return args[0]

    def chunk_step(s, t, kth):
        cb = s * CH + t
        c = row_v[pl.ds(cb * L, L)]
        cmax = _lane_max(c, iot)
        return lax.cond(cmax > kth, do_merge, skip, (kth, cb))

    def sup_step(s, kth):
        m = row_v[pl.ds(s * SUP, L)]
        for t in range(1, CH):
            m = jnp.maximum(m, row_v[pl.ds(s * SUP + t * L, L)])
        smax = _lane_max(m, iot)

        def hit(args):
            return lax.fori_loop(
                0, CH, lambda t, kk: chunk_step(args[1], t, kk), args[0])

        return lax.cond(smax > kth, hit, skip, (kth, s))

    lax.fori_loop(0, T2 // SUP, sup_step, neg)

    pltpu.sync_copy(vals_v, sh_vals.at[sid])
    pltpu.sync_copy(idx_v, sh_idx.at[sid])
    plsc.subcore_barrier()

    @pl.when(half == 0)
    def _():
        pltpu.sync_copy(sh_vals.at[sid + E // NC], pv_v)
        pltpu.sync_copy(sh_idx.at[sid + E // NC], pi_v)
        vals = vals_v[...]
        idxs = idx_v[...]
        av = lax.rev(pv_v[...], (0,))
        ai = lax.rev(pi_v[...], (0,))
        keep = _lex_gt(vals, idxs, av, ai)
        hv = jnp.where(keep, vals, av)
        hi = jnp.where(keep, idxs, ai)
        nv, ni = _bitonic_merge_desc(hv, hi, iot)

        topmask = iot < K
        ex = jnp.where(topmask, jnp.exp(nv - nv[0]), jnp.float32(0.0))
        ssum = ex
        for sh in (1, 2, 4, 8):
            ssum = ssum + jnp.take_along_axis(ssum, iot ^ sh, axis=0)
        vals_v[...] = ex / ssum
        idx_v[...] = ni
        pltpu.sync_copy(idx_v, idx_hbm.at[e])
        pltpu.sync_copy(vals_v, sh_vals.at[sid])
        pltpu.sync_copy(idx_v, sh_idx.at[sid])

    plsc.subcore_barrier()

    @pl.when(half == 1)
    def _():
        pltpu.sync_copy(sh_vals.at[sid - E // NC], vals_v)
        pltpu.sync_copy(sh_idx.at[sid - E // NC], idx_v)

    p = vals_v[...]
    idxs = idx_v[...]

    def zstep(i, carry):
        for u in range(8):
            out_v[pl.ds(i * 8 * L + u * L, L)] = jnp.zeros((L,), jnp.float32)
        return carry

    lax.fori_loop(0, T2 // (8 * L), zstep, 0)

    for k in range(K):
        tk = idxs[k]
        loc = tk - base_tok

        @pl.when((loc >= 0) & (loc < T2))
        def _(loc=loc, k=k):
            base = (loc >> 4) << 4
            ln = loc & (L - 1)
            vec = out_v[pl.ds(base, L)]
            out_v[pl.ds(base, L)] = jnp.where(iot == ln, p[k], vec)

    pltpu.sync_copy(out_v, router_hbm.at[e, pl.ds(base_tok, T2)])


def _sc_topk_call(noisy, interpret=False):
    mesh = plsc.VectorSubcoreMesh(
        core_axis_name="c", subcore_axis_name="s",
        num_cores=NC, num_subcores=NS)
    fn = pl.kernel(
        _sc_topk_body,
        out_type=[
            jax.ShapeDtypeStruct((E, T), jnp.float32),
            jax.ShapeDtypeStruct((E, L), jnp.int32),
        ],
        mesh=mesh,
        scratch_types=[
            pltpu.VMEM((T2,), jnp.float32),
            pltpu.VMEM((T2,), jnp.float32),
            pltpu.VMEM((L,), jnp.int32),
            pltpu.VMEM((L,), jnp.float32),
            pltpu.VMEM((L,), jnp.float32),
            pltpu.VMEM((L,), jnp.int32),
            pltpu.VMEM_SHARED((NS, L), jnp.float32),
            pltpu.VMEM_SHARED((NS, L), jnp.int32),
        ],
        interpret=interpret,
    )
    return fn(noisy)


def kernel(mh_output, W_route, b_route, W_noise, b_noise, eps):
    x = mh_output.reshape(T, D)
    Wc = jnp.concatenate([W_route, W_noise], axis=0)
    bc = jnp.concatenate([b_route, b_noise])[:, None]
    noisy = _noisy_call(x, Wc, bc, eps)
    router, idx16 = _sc_topk_call(noisy)
    return router, idx16[:, :K]
